# baseline (speedup 1.0000x reference)
import functools
import math

import jax
import jax.numpy as jnp
from jax import lax
from jax.experimental import pallas as pl
from jax.experimental.pallas import tpu as pltpu
from jax.experimental.pallas import tpu_sc as plsc

D_MODEL = 64
MAXLEN = 200
BATCH = 4096
NC, NS = 2, 16
NW = NC * NS
NSEQ = BATCH // NW
NPAIR = NSEQ // 2
SCALE = math.sqrt(D_MODEL)
WROW = MAXLEN // 2
N_WROWS = BATCH * WROW
VOCAB2 = 500000
GH = 104
POSPAD = 104

_mesh = plsc.VectorSubcoreMesh(core_axis_name="c", subcore_axis_name="s")


@functools.partial(
    pl.kernel,
    out_type=jax.ShapeDtypeStruct((BATCH, MAXLEN, D_MODEL), jnp.float32),
    mesh=_mesh,
    scratch_types=(
        [pltpu.VMEM((POSPAD, 2 * D_MODEL), jnp.float32)]
        + [pltpu.VMEM((MAXLEN,), jnp.int32)] * 2
        + [pltpu.VMEM((GH,), jnp.int32)] * 4
        + [pltpu.VMEM((GH, 2 * D_MODEL), jnp.float32)] * 4
        + [pltpu.VMEM((2 * MAXLEN, D_MODEL), jnp.float32)]
        + [pltpu.SemaphoreType.DMA] * 2
        + [pltpu.SemaphoreType.DMA] * 2
        + [pltpu.SemaphoreType.DMA] * 2
    ),
    compiler_params=pltpu.CompilerParams(use_tc_tiling_on_sc=True),
)
def _emb(table_hbm, x_hbm, pos_hbm, out_hbm, pos_v, *refs):
    idxb = refs[0:2]
    widxA = refs[2:4]
    widxB = refs[4:6]
    gA = refs[6:8]
    gB = refs[8:10]
    obuf = refs[10]
    sem_i = refs[11:13]
    sem_g = refs[13:15]
    sem_o = refs[15:17]

    wid = lax.axis_index("s") * NC + lax.axis_index("c")
    sbase = wid * NSEQ
    pbase = wid * NPAIR

    pltpu.sync_copy(pos_hbm, pos_v)

    def scale_pos(u, carry):
        for j in range(8):
            sl = pl.ds(j * 16, 16)
            pos_v[u, sl] = pos_v[u, sl] * SCALE
        return carry

    lax.fori_loop(0, POSPAD, scale_pos, 0)

    def start_idx(c, p):
        b = p % 2
        pltpu.async_copy(x_hbm.at[sbase + c], idxb[b], sem_i[b])

    def wait_idx(p):
        b = p % 2
        pltpu.make_async_copy(x_hbm.at[0], idxb[b], sem_i[b]).wait()

    def start_gather(p):
        b = p % 2
        src = idxb[b]
        for j in range(7):
            off = min(j * 16, 104 - 16)
            sl = pl.ds(off, 16)
            widxA[b][sl] = src[sl] >> 1
            widxB[b][sl] = src[pl.ds(min(96 + j * 16, MAXLEN - 16), 16)] >> 1
        pltpu.async_copy(table_hbm.at[widxA[b]], gA[b], sem_g[b])
        pltpu.async_copy(table_hbm.at[widxB[b]], gB[b], sem_g[b])

    def wait_gather(p):
        b = p % 2
        pltpu.make_async_copy(table_hbm.at[widxA[b]], gA[b], sem_g[b]).wait()
        pltpu.make_async_copy(table_hbm.at[widxB[b]], gB[b], sem_g[b]).wait()

    def start_out(c, p):
        b = p % 2
        pltpu.async_copy(obuf.at[pl.ds(b * MAXLEN, MAXLEN)],
                         out_hbm.at[sbase + c], sem_o[b])

    def wait_out(p):
        b = p % 2
        pltpu.make_async_copy(obuf.at[pl.ds(b * MAXLEN, MAXLEN)],
                              out_hbm.at[0], sem_o[b]).wait()

    def compute(p):
        ga, gb, src = gA[p % 2], gB[p % 2], idxb[p % 2]
        obase = (p % 2) * MAXLEN

        def do_token(iv, k, t, grow, gref):
            half = k % 2
            base = (iv[k] & 1) * D_MODEL
            for j in range(4):
                obuf[obase + t, pl.ds(j * 16, 16)] = (
                    gref[grow, pl.ds(base + j * 16, 16)] * SCALE
                    + pos_v[t // 2, pl.ds(half * D_MODEL + j * 16, 16)])

        def blkA(m, carry):
            t0 = m * 16
            iv = src[pl.ds(t0, 16)]
            for k in range(16):
                do_token(iv, k, t0 + k, t0 + k, ga)
            return carry

        def blkB(m, carry):
            off = lax.min(m * 16, MAXLEN - 16)
            iv = src[pl.ds(off, 16)]
            for k in range(16):
                do_token(iv, k, off + k, off - 96 + k, gb)
            return carry

        lax.fori_loop(0, 6, blkA, 0)
        lax.fori_loop(6, 13, blkB, 0)

    def step(c, p):
        @pl.when(c < NSEQ - 1)
        def _():
            wait_idx(p + 1)
            start_gather(p + 1)

        wait_gather(p)

        @pl.when(c >= 2)
        def _():
            wait_out(p)

        compute(p)
        start_out(c, p)

        @pl.when(c < NSEQ - 2)
        def _():
            start_idx(c + 2, p)

    start_idx(0, 0)
    start_idx(1, 1)
    wait_idx(0)
    start_gather(0)

    def main(i, carry):
        c0 = i * 2
        for k in range(2):
            step(c0 + k, k)
        return carry

    lax.fori_loop(0, NSEQ // 2, main, 0)

    for p in range(2):
        wait_out(p)


def kernel(x, token_table, pos_table):
    pos_p = jnp.pad(pos_table.reshape(WROW, 2 * D_MODEL),
                    ((0, POSPAD - WROW), (0, 0)))
    return _emb(token_table.reshape(VOCAB2, 2 * D_MODEL), x.astype(jnp.int32),
                pos_p)


# speedup vs baseline: 1.2932x; 1.2932x over previous
import functools
import math

import jax
import jax.numpy as jnp
from jax import lax
from jax.experimental import pallas as pl
from jax.experimental.pallas import tpu as pltpu
from jax.experimental.pallas import tpu_sc as plsc

D_MODEL = 64
MAXLEN = 200
BATCH = 4096
NC, NS = 2, 16
NW = NC * NS
NSEQ = BATCH // NW
SCALE = math.sqrt(D_MODEL)
NSLICE = D_MODEL // 16
NBUF = 4
LOOK = 2
G1, G2 = 104, 96

_mesh = plsc.VectorSubcoreMesh(core_axis_name="c", subcore_axis_name="s")


@functools.partial(
    pl.kernel,
    out_type=jax.ShapeDtypeStruct((BATCH, MAXLEN, D_MODEL), jnp.float32),
    mesh=_mesh,
    scratch_types=(
        [pltpu.VMEM((MAXLEN, D_MODEL), jnp.float32)]
        + [pltpu.VMEM((NSEQ, MAXLEN), jnp.int32)]
        + [pltpu.VMEM((MAXLEN, D_MODEL), jnp.float32)] * NBUF
        + [pltpu.SemaphoreType.DMA] * NBUF
        + [pltpu.SemaphoreType.DMA] * NBUF
    ),
    compiler_params=pltpu.CompilerParams(use_tc_tiling_on_sc=False),
)
def _emb(table_hbm, x_hbm, pos_hbm, out_hbm, pos_v, idx_v, *bufs_and_sems):
    rows = bufs_and_sems[:NBUF]
    sem_g = bufs_and_sems[NBUF:2 * NBUF]
    sem_o = bufs_and_sems[2 * NBUF:3 * NBUF]

    wid = lax.axis_index("s") * NC + lax.axis_index("c")
    sbase = wid * NSEQ

    pltpu.sync_copy(x_hbm.at[pl.ds(sbase, NSEQ)], idx_v)
    pltpu.sync_copy(pos_hbm, pos_v)

    def scale_pos(t, carry):
        for j in range(NSLICE):
            sl = pl.ds(j * 16, 16)
            pos_v[t, sl] = pos_v[t, sl] * SCALE
        return carry

    lax.fori_loop(0, MAXLEN, scale_pos, 0)

    def start_gather(c, b):
        pltpu.async_copy(table_hbm.at[idx_v.at[c, pl.ds(0, G1)]],
                         rows[b].at[pl.ds(0, G1)], sem_g[b])
        pltpu.async_copy(table_hbm.at[idx_v.at[c, pl.ds(G1, G2)]],
                         rows[b].at[pl.ds(G1, G2)], sem_g[b])

    def wait_gather(b):
        pltpu.make_async_copy(table_hbm.at[idx_v.at[0, pl.ds(0, G1)]],
                              rows[b].at[pl.ds(0, G1)], sem_g[b]).wait()
        pltpu.make_async_copy(table_hbm.at[idx_v.at[0, pl.ds(0, G2)]],
                              rows[b].at[pl.ds(G1, G2)], sem_g[b]).wait()

    def start_out(c, b):
        pltpu.async_copy(rows[b], out_hbm.at[sbase + c], sem_o[b])

    def wait_out(b):
        pltpu.make_async_copy(rows[b], out_hbm.at[0], sem_o[b]).wait()

    def compute(b):
        buf = rows[b]

        def row_body(t, carry):
            for j in range(NSLICE):
                sl = pl.ds(j * 16, 16)
                buf[t, sl] = buf[t, sl] * SCALE + pos_v[t, sl]
            return carry

        lax.fori_loop(0, MAXLEN, row_body, 0)

    start_gather(0, 0)
    start_gather(1, 1)

    for c in range(2):
        start_gather(c + LOOK, (c + LOOK) % NBUF)
        wait_gather(c % NBUF)
        compute(c % NBUF)
        start_out(c, c % NBUF)
    for c in range(2, 4):
        wait_out((c + LOOK) % NBUF)
        start_gather(c + LOOK, (c + LOOK) % NBUF)
        wait_gather(c % NBUF)
        compute(c % NBUF)
        start_out(c, c % NBUF)

    def steady(i, carry):
        c0 = i * NBUF
        for k in range(NBUF):
            c = c0 + k
            wait_out((k + LOOK) % NBUF)
            start_gather(c + LOOK, (k + LOOK) % NBUF)
            wait_gather(k)
            compute(k)
            start_out(c, k)
        return carry

    lax.fori_loop(1, NSEQ // NBUF - 1, steady, 0)

    for c in range(NSEQ - 4, NSEQ - 2):
        b = c % NBUF
        wait_out((b + LOOK) % NBUF)
        start_gather(c + LOOK, (b + LOOK) % NBUF)
        wait_gather(b)
        compute(b)
        start_out(c, b)
    for c in range(NSEQ - 2, NSEQ):
        b = c % NBUF
        wait_gather(b)
        compute(b)
        start_out(c, b)
    for b in range(NBUF):
        wait_out(b)


def kernel(x, token_table, pos_table):
    return _emb(token_table, x.astype(jnp.int32), pos_table)
